# Initial kernel scaffold; baseline (speedup 1.0000x reference)
import jax
import jax.numpy as jnp
from jax.experimental import pallas as pl


def kernel(x, edge_index, W_l, b_l, W_r):
    raise NotImplementedError("write your pallas kernel here")


# speedup vs baseline: 6.0089x; 6.0089x over previous
import functools

import jax
import jax.numpy as jnp
from jax import lax
from jax.experimental import pallas as pl
from jax.experimental.pallas import tpu as pltpu
from jax.experimental.pallas import tpu_sc as plsc

N_NODES = 10000
N_EDGES = 320000
D = 128

NC = 2
NS = 16
NW = NC * NS
EPW = N_EDGES // NW
K = 80
NCHUNK = EPW // K
N_PAD = 10240
RPT = N_PAD // NS
CW = 16


def _sc_aggregate(src3, dst3, x, ones_h, z_acc, z_cnt):
  mesh = plsc.VectorSubcoreMesh(core_axis_name="c", subcore_axis_name="s")

  @functools.partial(
      pl.kernel,
      out_type=[
          jax.ShapeDtypeStruct((NC, N_PAD, D), jnp.float32),
          jax.ShapeDtypeStruct((NC, N_PAD, CW), jnp.float32),
      ],
      mesh=mesh,
      compiler_params=pltpu.CompilerParams(use_tc_tiling_on_sc=False),
      scratch_types=[
          pltpu.VMEM((K,), jnp.int32),
          pltpu.VMEM((K,), jnp.int32),
          pltpu.VMEM((K, D), jnp.float32),
          pltpu.VMEM((K, CW), jnp.float32),
          pltpu.VMEM_SHARED((N_PAD, D), jnp.float32),
          pltpu.VMEM_SHARED((N_PAD, CW), jnp.float32),
          pltpu.SemaphoreType.DMA,
      ],
  )
  def agg_kernel(src_h, dst_h, x_h, ones_hbm, zacc_h, zcnt_h,
                 part_o, cnt_o,
                 src_v, dst_v, rows_v, ones_v, acc, cnt, sem):
    c = lax.axis_index("c")
    s = lax.axis_index("s")
    wid = s * NC + c

    pltpu.sync_copy(ones_hbm, ones_v)
    pltpu.sync_copy(zacc_h, acc.at[pl.ds(s * RPT, RPT)])
    pltpu.sync_copy(zcnt_h, cnt.at[pl.ds(s * RPT, RPT)])
    plsc.subcore_barrier()

    def chunk(i, carry):
      base = wid * EPW + i * K
      pltpu.sync_copy(src_h.at[pl.ds(base, K)], src_v)
      pltpu.sync_copy(dst_h.at[pl.ds(base, K)], dst_v)
      pltpu.async_copy(x_h.at[src_v], rows_v, sem).wait()
      pltpu.sync_copy(rows_v, acc.at[dst_v], add=True)
      pltpu.sync_copy(ones_v, cnt.at[dst_v], add=True)
      return carry

    lax.fori_loop(0, NCHUNK, chunk, 0)

    plsc.subcore_barrier()
    pltpu.sync_copy(acc.at[pl.ds(s * RPT, RPT)],
                    part_o.at[c, pl.ds(s * RPT, RPT)])
    pltpu.sync_copy(cnt.at[pl.ds(s * RPT, RPT)],
                    cnt_o.at[c, pl.ds(s * RPT, RPT)])

  return agg_kernel(src3, dst3, x, ones_h, z_acc, z_cnt)


def _tc_dense(part, cnt, x, W_l, b_l, W_r):
  R = 1000

  def body(part_ref, cnt_ref, x_ref, wl_ref, bl_ref, wr_ref, o_ref):
    p = part_ref[0] + part_ref[1]
    deg = cnt_ref[0, :, 0:1] + cnt_ref[1, :, 0:1]
    agg = p / jnp.maximum(deg, 1.0)
    xb = x_ref[...]
    dn = (((1,), (1,)), ((), ()))
    h = (lax.dot_general(agg, wl_ref[...], dn,
                         preferred_element_type=jnp.float32)
         + bl_ref[...]
         + lax.dot_general(xb, wr_ref[...], dn,
                           preferred_element_type=jnp.float32))
    o_ref[...] = xb + jnp.maximum(h, 0.0)

  return pl.pallas_call(
      body,
      grid=(N_NODES // R,),
      in_specs=[
          pl.BlockSpec((NC, R, D), lambda i: (0, i, 0)),
          pl.BlockSpec((NC, R, CW), lambda i: (0, i, 0)),
          pl.BlockSpec((R, D), lambda i: (i, 0)),
          pl.BlockSpec((D, D), lambda i: (0, 0)),
          pl.BlockSpec((1, D), lambda i: (0, 0)),
          pl.BlockSpec((D, D), lambda i: (0, 0)),
      ],
      out_specs=pl.BlockSpec((R, D), lambda i: (i, 0)),
      out_shape=jax.ShapeDtypeStruct((N_NODES, D), jnp.float32),
  )(part, cnt, x, W_l, b_l, W_r)


@jax.jit
def kernel(x, edge_index, W_l, b_l, W_r):
  src1 = edge_index[0].astype(jnp.int32)
  dst1 = edge_index[1].astype(jnp.int32)
  ones_h = jnp.ones((K, CW), jnp.float32)
  z_acc = jnp.zeros((RPT, D), jnp.float32)
  z_cnt = jnp.zeros((RPT, CW), jnp.float32)
  part, cnt = _sc_aggregate(src1, dst1, x, ones_h, z_acc, z_cnt)
  return _tc_dense(part, cnt, x, W_l, b_l.reshape(1, D), W_r)


# speedup vs baseline: 10.6312x; 1.7692x over previous
import functools

import jax
import jax.numpy as jnp
from jax import lax
from jax.experimental import pallas as pl
from jax.experimental.pallas import tpu as pltpu
from jax.experimental.pallas import tpu_sc as plsc

N_NODES = 10000
N_EDGES = 320000
D = 128

NC = 2
NS = 16
NW = NC * NS
EPW = N_EDGES // NW
K = 80
NCHUNK = EPW // K
N_PAD = 10240
RPT = N_PAD // NS
CW = 16


def _sc_aggregate(src1, dst1, x, ones_h, z_acc, z_cnt):
  mesh = plsc.VectorSubcoreMesh(core_axis_name="c", subcore_axis_name="s")

  @functools.partial(
      pl.kernel,
      out_type=[
          jax.ShapeDtypeStruct((NC, N_PAD, D), jnp.float32),
          jax.ShapeDtypeStruct((NC, N_PAD, CW), jnp.float32),
      ],
      mesh=mesh,
      compiler_params=pltpu.CompilerParams(use_tc_tiling_on_sc=False),
      scratch_types=[
          pltpu.VMEM((K,), jnp.int32),
          pltpu.VMEM((K,), jnp.int32),
          pltpu.VMEM((K,), jnp.int32),
          pltpu.VMEM((K,), jnp.int32),
          pltpu.VMEM((K, D), jnp.float32),
          pltpu.VMEM((K, D), jnp.float32),
          pltpu.VMEM((K, CW), jnp.float32),
          pltpu.VMEM_SHARED((N_PAD, D), jnp.float32),
          pltpu.VMEM_SHARED((N_PAD, CW), jnp.float32),
          pltpu.SemaphoreType.DMA,
          pltpu.SemaphoreType.DMA,
          pltpu.SemaphoreType.DMA,
          pltpu.SemaphoreType.DMA,
          pltpu.SemaphoreType.DMA,
          pltpu.SemaphoreType.DMA,
          pltpu.SemaphoreType.DMA,
          pltpu.SemaphoreType.DMA,
          pltpu.SemaphoreType.DMA,
          pltpu.SemaphoreType.DMA,
      ],
  )
  def agg_kernel(src_h, dst_h, x_h, ones_hbm, zacc_h, zcnt_h,
                 part_o, cnt_o,
                 sv0, sv1, dv0, dv1, r0, r1, ones_v, acc, cnt,
                 gs0, gs1, ss0, ss1, cs0, cs1, is0, is1, id0, id1):
    sv = [sv0, sv1]
    dv = [dv0, dv1]
    rows = [r0, r1]
    gs = [gs0, gs1]
    ss = [ss0, ss1]
    cs = [cs0, cs1]
    isem = [is0, is1]
    idsem = [id0, id1]

    c_ax = lax.axis_index("c")
    s_ax = lax.axis_index("s")
    wid = s_ax * NC + c_ax
    ebase = wid * EPW

    def src_slice(c):
      return src_h.at[pl.ds(ebase + c * K, K)]

    def dst_slice(c):
      return dst_h.at[pl.ds(ebase + c * K, K)]

    pltpu.sync_copy(ones_hbm, ones_v)
    pltpu.sync_copy(zacc_h, acc.at[pl.ds(s_ax * RPT, RPT)])
    pltpu.sync_copy(zcnt_h, cnt.at[pl.ds(s_ax * RPT, RPT)])
    plsc.subcore_barrier()

    pltpu.sync_copy(src_slice(0), sv[0])
    pltpu.async_copy(x_h.at[sv[0]], rows[0], gs[0])
    pltpu.async_copy(dst_slice(0), dv[0], idsem[0])
    pltpu.async_copy(src_slice(1), sv[1], isem[1])
    pltpu.make_async_copy(x_h.at[sv[0]], rows[0], gs[0]).wait()
    pltpu.make_async_copy(dst_slice(0), dv[0], idsem[0]).wait()
    pltpu.async_copy(rows[0], acc.at[dv[0]], ss[0], add=True)
    pltpu.async_copy(ones_v, cnt.at[dv[0]], cs[0], add=True)
    pltpu.async_copy(dst_slice(1), dv[1], idsem[1])
    pltpu.async_copy(src_slice(2), sv[0], isem[0])
    pltpu.make_async_copy(src_slice(1), sv[1], isem[1]).wait()
    pltpu.async_copy(x_h.at[sv[1]], rows[1], gs[1])

    @pl.loop(1, NCHUNK, step=2)
    def _body(g):
      for db in (0, 1):
        c = g + db
        b = 1 - db
        bp = db
        pltpu.make_async_copy(x_h.at[sv[b]], rows[b], gs[b]).wait()

        @pl.when(c + 2 < NCHUNK)
        def _():
          pltpu.async_copy(src_slice(c + 2), sv[b], isem[b])

        pltpu.make_async_copy(dst_slice(c), dv[b], idsem[b]).wait()
        pltpu.async_copy(rows[b], acc.at[dv[b]], ss[b], add=True)
        pltpu.async_copy(ones_v, cnt.at[dv[b]], cs[b], add=True)
        pltpu.make_async_copy(rows[bp], acc.at[dv[bp]], ss[bp]).wait()
        pltpu.make_async_copy(ones_v, cnt.at[dv[bp]], cs[bp]).wait()

        @pl.when(c + 1 < NCHUNK)
        def _():
          pltpu.async_copy(dst_slice(c + 1), dv[bp], idsem[bp])
          pltpu.make_async_copy(src_slice(c + 1), sv[bp], isem[bp]).wait()
          pltpu.async_copy(x_h.at[sv[bp]], rows[bp], gs[bp])

    pltpu.make_async_copy(rows[0], acc.at[dv[0]], ss[0]).wait()
    pltpu.make_async_copy(ones_v, cnt.at[dv[0]], cs[0]).wait()

    plsc.subcore_barrier()
    pltpu.sync_copy(acc.at[pl.ds(s_ax * RPT, RPT)],
                    part_o.at[c_ax, pl.ds(s_ax * RPT, RPT)])
    pltpu.sync_copy(cnt.at[pl.ds(s_ax * RPT, RPT)],
                    cnt_o.at[c_ax, pl.ds(s_ax * RPT, RPT)])

  return agg_kernel(src1, dst1, x, ones_h, z_acc, z_cnt)


def _tc_dense(part, cnt, x, W_l, b_l, W_r):
  R = 1000

  def body(part_ref, cnt_ref, x_ref, wl_ref, bl_ref, wr_ref, o_ref):
    p = part_ref[0] + part_ref[1]
    deg = cnt_ref[0, :, 0:1] + cnt_ref[1, :, 0:1]
    agg = p / jnp.maximum(deg, 1.0)
    xb = x_ref[...]
    dn = (((1,), (1,)), ((), ()))
    h = (lax.dot_general(agg, wl_ref[...], dn,
                         preferred_element_type=jnp.float32)
         + bl_ref[...]
         + lax.dot_general(xb, wr_ref[...], dn,
                           preferred_element_type=jnp.float32))
    o_ref[...] = xb + jnp.maximum(h, 0.0)

  return pl.pallas_call(
      body,
      grid=(N_NODES // R,),
      in_specs=[
          pl.BlockSpec((NC, R, D), lambda i: (0, i, 0)),
          pl.BlockSpec((NC, R, CW), lambda i: (0, i, 0)),
          pl.BlockSpec((R, D), lambda i: (i, 0)),
          pl.BlockSpec((D, D), lambda i: (0, 0)),
          pl.BlockSpec((1, D), lambda i: (0, 0)),
          pl.BlockSpec((D, D), lambda i: (0, 0)),
      ],
      out_specs=pl.BlockSpec((R, D), lambda i: (i, 0)),
      out_shape=jax.ShapeDtypeStruct((N_NODES, D), jnp.float32),
  )(part, cnt, x, W_l, b_l, W_r)


@jax.jit
def kernel(x, edge_index, W_l, b_l, W_r):
  src1 = edge_index[0].astype(jnp.int32)
  dst1 = edge_index[1].astype(jnp.int32)
  ones_h = jnp.ones((K, CW), jnp.float32)
  z_acc = jnp.zeros((RPT, D), jnp.float32)
  z_cnt = jnp.zeros((RPT, CW), jnp.float32)
  part, cnt = _sc_aggregate(src1, dst1, x, ones_h, z_acc, z_cnt)
  return _tc_dense(part, cnt, x, W_l, b_l.reshape(1, D), W_r)


# speedup vs baseline: 12.1947x; 1.1471x over previous
import functools

import jax
import jax.numpy as jnp
from jax import lax
from jax.experimental import pallas as pl
from jax.experimental.pallas import tpu as pltpu
from jax.experimental.pallas import tpu_sc as plsc

N_NODES = 10000
N_EDGES = 320000
D = 128

NC = 2
NS = 16
NW = NC * NS
EPW = N_EDGES // NW
K = 128
NCHUNK = EPW // K
KT = EPW - NCHUNK * K
N_PAD = 10240
RPT = N_PAD // NS
CW = 16


def _sc_aggregate(src1, dst1, x, ones_h, z_acc, z_cnt):
  mesh = plsc.VectorSubcoreMesh(core_axis_name="c", subcore_axis_name="s")

  @functools.partial(
      pl.kernel,
      out_type=[
          jax.ShapeDtypeStruct((NC, N_PAD, D), jnp.float32),
          jax.ShapeDtypeStruct((NC, N_PAD, CW), jnp.float32),
      ],
      mesh=mesh,
      compiler_params=pltpu.CompilerParams(use_tc_tiling_on_sc=False),
      scratch_types=[
          pltpu.VMEM((K,), jnp.int32),
          pltpu.VMEM((K,), jnp.int32),
          pltpu.VMEM((K,), jnp.int32),
          pltpu.VMEM((K,), jnp.int32),
          pltpu.VMEM((KT,), jnp.int32),
          pltpu.VMEM((KT,), jnp.int32),
          pltpu.VMEM((K, D), jnp.float32),
          pltpu.VMEM((K, D), jnp.float32),
          pltpu.VMEM((K, CW), jnp.float32),
          pltpu.VMEM_SHARED((N_PAD, D), jnp.float32),
          pltpu.VMEM_SHARED((N_PAD, CW), jnp.float32),
          pltpu.SemaphoreType.DMA,
          pltpu.SemaphoreType.DMA,
          pltpu.SemaphoreType.DMA,
          pltpu.SemaphoreType.DMA,
          pltpu.SemaphoreType.DMA,
          pltpu.SemaphoreType.DMA,
          pltpu.SemaphoreType.DMA,
          pltpu.SemaphoreType.DMA,
          pltpu.SemaphoreType.DMA,
          pltpu.SemaphoreType.DMA,
      ],
  )
  def agg_kernel(src_h, dst_h, x_h, ones_hbm, zacc_h, zcnt_h,
                 part_o, cnt_o,
                 sv0, sv1, dv0, dv1, svt, dvt, r0, r1, ones_v, acc, cnt,
                 gs0, gs1, ss0, ss1, cs0, cs1, is0, is1, id0, id1):
    sv = [sv0, sv1]
    dv = [dv0, dv1]
    rows = [r0, r1]
    gs = [gs0, gs1]
    ss = [ss0, ss1]
    cs = [cs0, cs1]
    isem = [is0, is1]
    idsem = [id0, id1]

    c_ax = lax.axis_index("c")
    s_ax = lax.axis_index("s")
    wid = s_ax * NC + c_ax
    ebase = wid * EPW

    def src_slice(c):
      return src_h.at[pl.ds(ebase + c * K, K)]

    def dst_slice(c):
      return dst_h.at[pl.ds(ebase + c * K, K)]

    pltpu.sync_copy(ones_hbm, ones_v)
    pltpu.sync_copy(zacc_h, acc.at[pl.ds(s_ax * RPT, RPT)])
    pltpu.sync_copy(zcnt_h, cnt.at[pl.ds(s_ax * RPT, RPT)])
    plsc.subcore_barrier()

    pltpu.sync_copy(src_slice(0), sv[0])
    pltpu.async_copy(x_h.at[sv[0]], rows[0], gs[0])
    pltpu.async_copy(dst_slice(0), dv[0], idsem[0])
    pltpu.async_copy(src_slice(1), sv[1], isem[1])
    pltpu.make_async_copy(x_h.at[sv[0]], rows[0], gs[0]).wait()
    pltpu.make_async_copy(dst_slice(0), dv[0], idsem[0]).wait()
    pltpu.async_copy(rows[0], acc.at[dv[0]], ss[0], add=True)
    pltpu.async_copy(ones_v, cnt.at[dv[0]], cs[0], add=True)
    pltpu.async_copy(dst_slice(1), dv[1], idsem[1])
    pltpu.async_copy(src_slice(2), sv[0], isem[0])
    pltpu.make_async_copy(src_slice(1), sv[1], isem[1]).wait()
    pltpu.async_copy(x_h.at[sv[1]], rows[1], gs[1])

    def chunk_step(c, b, bp):
      pltpu.make_async_copy(x_h.at[sv[b]], rows[b], gs[b]).wait()

      def _prefetch_src():
        pltpu.async_copy(src_slice(c + 2), sv[b], isem[b])

      if isinstance(c, int):
        if c + 2 < NCHUNK:
          _prefetch_src()
      else:
        pl.when(c + 2 < NCHUNK)(_prefetch_src)

      pltpu.make_async_copy(dst_slice(c), dv[b], idsem[b]).wait()
      pltpu.async_copy(rows[b], acc.at[dv[b]], ss[b], add=True)
      pltpu.async_copy(ones_v, cnt.at[dv[b]], cs[b], add=True)
      pltpu.make_async_copy(rows[bp], acc.at[dv[bp]], ss[bp]).wait()
      pltpu.make_async_copy(ones_v, cnt.at[dv[bp]], cs[bp]).wait()

      def _next_gather():
        pltpu.async_copy(dst_slice(c + 1), dv[bp], idsem[bp])
        pltpu.make_async_copy(src_slice(c + 1), sv[bp], isem[bp]).wait()
        pltpu.async_copy(x_h.at[sv[bp]], rows[bp], gs[bp])

      if isinstance(c, int):
        if c + 1 < NCHUNK:
          _next_gather()
      else:
        pl.when(c + 1 < NCHUNK)(_next_gather)

    chunk_step(1, 1, 0)

    @pl.loop(2, NCHUNK, step=2)
    def _body(g):
      for db in (0, 1):
        chunk_step(g + db, db, 1 - db)

    tbase = ebase + NCHUNK * K
    pltpu.sync_copy(src_h.at[pl.ds(tbase, KT)], svt)
    pltpu.sync_copy(dst_h.at[pl.ds(tbase, KT)], dvt)
    rows_t = rows[0].at[pl.ds(0, KT)]
    ones_t = ones_v.at[pl.ds(0, KT)]
    pltpu.async_copy(x_h.at[svt], rows_t, gs[0]).wait()
    pltpu.async_copy(rows_t, acc.at[dvt], ss[0], add=True)
    pltpu.async_copy(ones_t, cnt.at[dvt], cs[0], add=True)
    pltpu.make_async_copy(rows_t, acc.at[dvt], ss[0]).wait()
    pltpu.make_async_copy(ones_t, cnt.at[dvt], cs[0]).wait()
    pltpu.make_async_copy(rows[1], acc.at[dv[1]], ss[1]).wait()
    pltpu.make_async_copy(ones_v, cnt.at[dv[1]], cs[1]).wait()

    plsc.subcore_barrier()
    pltpu.sync_copy(acc.at[pl.ds(s_ax * RPT, RPT)],
                    part_o.at[c_ax, pl.ds(s_ax * RPT, RPT)])
    pltpu.sync_copy(cnt.at[pl.ds(s_ax * RPT, RPT)],
                    cnt_o.at[c_ax, pl.ds(s_ax * RPT, RPT)])

  return agg_kernel(src1, dst1, x, ones_h, z_acc, z_cnt)


def _tc_dense(part, cnt, x, W_l, b_l, W_r):
  R = 1000

  def body(part_ref, cnt_ref, x_ref, wl_ref, bl_ref, wr_ref, o_ref):
    p = part_ref[0] + part_ref[1]
    deg = cnt_ref[0, :, 0:1] + cnt_ref[1, :, 0:1]
    agg = p / jnp.maximum(deg, 1.0)
    xb = x_ref[...]
    dn = (((1,), (1,)), ((), ()))
    h = (lax.dot_general(agg, wl_ref[...], dn,
                         preferred_element_type=jnp.float32)
         + bl_ref[...]
         + lax.dot_general(xb, wr_ref[...], dn,
                           preferred_element_type=jnp.float32))
    o_ref[...] = xb + jnp.maximum(h, 0.0)

  return pl.pallas_call(
      body,
      grid=(N_NODES // R,),
      in_specs=[
          pl.BlockSpec((NC, R, D), lambda i: (0, i, 0)),
          pl.BlockSpec((NC, R, CW), lambda i: (0, i, 0)),
          pl.BlockSpec((R, D), lambda i: (i, 0)),
          pl.BlockSpec((D, D), lambda i: (0, 0)),
          pl.BlockSpec((1, D), lambda i: (0, 0)),
          pl.BlockSpec((D, D), lambda i: (0, 0)),
      ],
      out_specs=pl.BlockSpec((R, D), lambda i: (i, 0)),
      out_shape=jax.ShapeDtypeStruct((N_NODES, D), jnp.float32),
  )(part, cnt, x, W_l, b_l, W_r)


@jax.jit
def kernel(x, edge_index, W_l, b_l, W_r):
  src1 = edge_index[0].astype(jnp.int32)
  dst1 = edge_index[1].astype(jnp.int32)
  ones_h = jnp.ones((K, CW), jnp.float32)
  z_acc = jnp.zeros((RPT, D), jnp.float32)
  z_cnt = jnp.zeros((RPT, CW), jnp.float32)
  part, cnt = _sc_aggregate(src1, dst1, x, ones_h, z_acc, z_cnt)
  return _tc_dense(part, cnt, x, W_l, b_l.reshape(1, D), W_r)


# speedup vs baseline: 12.2090x; 1.0012x over previous
import functools

import jax
import jax.numpy as jnp
from jax import lax
from jax.experimental import pallas as pl
from jax.experimental.pallas import tpu as pltpu
from jax.experimental.pallas import tpu_sc as plsc

N_NODES = 10000
N_EDGES = 320000
D = 128

NC = 2
NS = 16
NW = NC * NS
EPW = N_EDGES // NW
K = 128
NCHUNK = EPW // K
KT = EPW - NCHUNK * K
N_PAD = 10240
RPT = N_PAD // NS
CW = 16


def _sc_aggregate(src1, dst1, x, ones_h, z_acc, z_cnt):
  mesh = plsc.VectorSubcoreMesh(core_axis_name="c", subcore_axis_name="s")

  @functools.partial(
      pl.kernel,
      out_type=[
          jax.ShapeDtypeStruct((NC, N_PAD, D), jnp.float32),
          jax.ShapeDtypeStruct((NC, N_PAD, CW), jnp.float32),
      ],
      mesh=mesh,
      compiler_params=pltpu.CompilerParams(use_tc_tiling_on_sc=False),
      scratch_types=[
          pltpu.VMEM((K,), jnp.int32),
          pltpu.VMEM((K,), jnp.int32),
          pltpu.VMEM((K,), jnp.int32),
          pltpu.VMEM((K,), jnp.int32),
          pltpu.VMEM((KT,), jnp.int32),
          pltpu.VMEM((KT,), jnp.int32),
          pltpu.VMEM((K, D), jnp.float32),
          pltpu.VMEM((K, D), jnp.float32),
          pltpu.VMEM((K, CW), jnp.float32),
          pltpu.VMEM_SHARED((N_PAD, D), jnp.float32),
          pltpu.VMEM_SHARED((N_PAD, CW), jnp.float32),
          pltpu.SemaphoreType.DMA,
          pltpu.SemaphoreType.DMA,
          pltpu.SemaphoreType.DMA,
          pltpu.SemaphoreType.DMA,
          pltpu.SemaphoreType.DMA,
          pltpu.SemaphoreType.DMA,
          pltpu.SemaphoreType.DMA,
          pltpu.SemaphoreType.DMA,
          pltpu.SemaphoreType.DMA,
          pltpu.SemaphoreType.DMA,
      ],
  )
  def agg_kernel(src_h, dst_h, x_h, ones_hbm, zacc_h, zcnt_h,
                 part_o, cnt_o,
                 sv0, sv1, dv0, dv1, svt, dvt, r0, r1, ones_v, acc, cnt,
                 gs0, gs1, ss0, ss1, cs0, cs1, is0, is1, id0, id1):
    sv = [sv0, sv1]
    dv = [dv0, dv1]
    rows = [r0, r1]
    gs = [gs0, gs1]
    ss = [ss0, ss1]
    cs = [cs0, cs1]
    isem = [is0, is1]
    idsem = [id0, id1]

    c_ax = lax.axis_index("c")
    s_ax = lax.axis_index("s")
    wid = s_ax * NC + c_ax
    ebase = wid * EPW

    def src_slice(c):
      return src_h.at[pl.ds(ebase + c * K, K)]

    def dst_slice(c):
      return dst_h.at[pl.ds(ebase + c * K, K)]

    pltpu.sync_copy(src_slice(0), sv[0])
    pltpu.async_copy(x_h.at[sv[0]], rows[0], gs[0])
    pltpu.async_copy(dst_slice(0), dv[0], idsem[0])
    pltpu.async_copy(src_slice(1), sv[1], isem[1])
    pltpu.sync_copy(ones_hbm, ones_v)
    pltpu.sync_copy(zacc_h, acc.at[pl.ds(s_ax * RPT, RPT)])
    pltpu.sync_copy(zcnt_h, cnt.at[pl.ds(s_ax * RPT, RPT)])
    plsc.subcore_barrier()
    pltpu.make_async_copy(x_h.at[sv[0]], rows[0], gs[0]).wait()
    pltpu.make_async_copy(dst_slice(0), dv[0], idsem[0]).wait()
    pltpu.async_copy(rows[0], acc.at[dv[0]], ss[0], add=True)
    pltpu.async_copy(ones_v, cnt.at[dv[0]], cs[0], add=True)
    pltpu.async_copy(dst_slice(1), dv[1], idsem[1])
    pltpu.async_copy(src_slice(2), sv[0], isem[0])
    pltpu.make_async_copy(src_slice(1), sv[1], isem[1]).wait()
    pltpu.async_copy(x_h.at[sv[1]], rows[1], gs[1])

    def chunk_step(c, b, bp):
      pltpu.make_async_copy(x_h.at[sv[b]], rows[b], gs[b]).wait()

      def _prefetch_src():
        pltpu.async_copy(src_slice(c + 2), sv[b], isem[b])

      if isinstance(c, int):
        if c + 2 < NCHUNK:
          _prefetch_src()
      else:
        pl.when(c + 2 < NCHUNK)(_prefetch_src)

      pltpu.make_async_copy(dst_slice(c), dv[b], idsem[b]).wait()
      pltpu.async_copy(rows[b], acc.at[dv[b]], ss[b], add=True)
      pltpu.async_copy(ones_v, cnt.at[dv[b]], cs[b], add=True)
      pltpu.make_async_copy(rows[bp], acc.at[dv[bp]], ss[bp]).wait()
      pltpu.make_async_copy(ones_v, cnt.at[dv[bp]], cs[bp]).wait()

      def _next_gather():
        pltpu.async_copy(dst_slice(c + 1), dv[bp], idsem[bp])
        pltpu.make_async_copy(src_slice(c + 1), sv[bp], isem[bp]).wait()
        pltpu.async_copy(x_h.at[sv[bp]], rows[bp], gs[bp])

      if isinstance(c, int):
        if c + 1 < NCHUNK:
          _next_gather()
      else:
        pl.when(c + 1 < NCHUNK)(_next_gather)

    chunk_step(1, 1, 0)

    @pl.loop(2, NCHUNK, step=2)
    def _body(g):
      for db in (0, 1):
        chunk_step(g + db, db, 1 - db)

    tbase = ebase + NCHUNK * K
    pltpu.sync_copy(src_h.at[pl.ds(tbase, KT)], svt)
    pltpu.sync_copy(dst_h.at[pl.ds(tbase, KT)], dvt)
    rows_t = rows[0].at[pl.ds(0, KT)]
    ones_t = ones_v.at[pl.ds(0, KT)]
    pltpu.async_copy(x_h.at[svt], rows_t, gs[0]).wait()
    pltpu.async_copy(rows_t, acc.at[dvt], ss[0], add=True)
    pltpu.async_copy(ones_t, cnt.at[dvt], cs[0], add=True)
    pltpu.make_async_copy(rows_t, acc.at[dvt], ss[0]).wait()
    pltpu.make_async_copy(ones_t, cnt.at[dvt], cs[0]).wait()
    pltpu.make_async_copy(rows[1], acc.at[dv[1]], ss[1]).wait()
    pltpu.make_async_copy(ones_v, cnt.at[dv[1]], cs[1]).wait()

    plsc.subcore_barrier()
    pltpu.sync_copy(acc.at[pl.ds(s_ax * RPT, RPT)],
                    part_o.at[c_ax, pl.ds(s_ax * RPT, RPT)])
    pltpu.sync_copy(cnt.at[pl.ds(s_ax * RPT, RPT)],
                    cnt_o.at[c_ax, pl.ds(s_ax * RPT, RPT)])

  return agg_kernel(src1, dst1, x, ones_h, z_acc, z_cnt)


def _tc_root(x, W_r):
  R = 1000

  def body(x_ref, wr_ref, o_ref):
    dn = (((1,), (1,)), ((), ()))
    o_ref[...] = lax.dot_general(x_ref[...], wr_ref[...], dn,
                                 preferred_element_type=jnp.float32)

  return pl.pallas_call(
      body,
      grid=(N_NODES // R,),
      in_specs=[
          pl.BlockSpec((R, D), lambda i: (i, 0)),
          pl.BlockSpec((D, D), lambda i: (0, 0)),
      ],
      out_specs=pl.BlockSpec((R, D), lambda i: (i, 0)),
      out_shape=jax.ShapeDtypeStruct((N_NODES, D), jnp.float32),
  )(x, W_r)


def _tc_combine(part, cnt, x, root, W_l, b_l):
  R = 1000

  def body(part_ref, cnt_ref, x_ref, root_ref, wl_ref, bl_ref, o_ref):
    p = part_ref[0] + part_ref[1]
    deg = cnt_ref[0, :, 0:1] + cnt_ref[1, :, 0:1]
    agg = p / jnp.maximum(deg, 1.0)
    xb = x_ref[...]
    dn = (((1,), (1,)), ((), ()))
    h = (lax.dot_general(agg, wl_ref[...], dn,
                         preferred_element_type=jnp.float32)
         + bl_ref[...]
         + root_ref[...])
    o_ref[...] = xb + jnp.maximum(h, 0.0)

  return pl.pallas_call(
      body,
      grid=(N_NODES // R,),
      in_specs=[
          pl.BlockSpec((NC, R, D), lambda i: (0, i, 0)),
          pl.BlockSpec((NC, R, CW), lambda i: (0, i, 0)),
          pl.BlockSpec((R, D), lambda i: (i, 0)),
          pl.BlockSpec((R, D), lambda i: (i, 0)),
          pl.BlockSpec((D, D), lambda i: (0, 0)),
          pl.BlockSpec((1, D), lambda i: (0, 0)),
      ],
      out_specs=pl.BlockSpec((R, D), lambda i: (i, 0)),
      out_shape=jax.ShapeDtypeStruct((N_NODES, D), jnp.float32),
  )(part, cnt, x, root, W_l, b_l)


@jax.jit
def kernel(x, edge_index, W_l, b_l, W_r):
  src1 = edge_index[0].astype(jnp.int32)
  dst1 = edge_index[1].astype(jnp.int32)
  ones_h = jnp.ones((K, CW), jnp.float32)
  z_acc = jnp.zeros((RPT, D), jnp.float32)
  z_cnt = jnp.zeros((RPT, CW), jnp.float32)
  root = _tc_root(x, W_r)
  part, cnt = _sc_aggregate(src1, dst1, x, ones_h, z_acc, z_cnt)
  return _tc_combine(part, cnt, x, root, W_l, b_l.reshape(1, D))


# speedup vs baseline: 12.9269x; 1.0588x over previous
import functools

import jax
import jax.numpy as jnp
from jax import lax
from jax.experimental import pallas as pl
from jax.experimental.pallas import tpu as pltpu
from jax.experimental.pallas import tpu_sc as plsc

N_NODES = 10000
N_EDGES = 320000
D = 128

NC = 2
NS = 16
NW = NC * NS
EPW = N_EDGES // NW
K = 128
NCHUNK = EPW // K
KT = EPW - NCHUNK * K
N_PAD = 10240
RPT = N_PAD // NS
CW = 16


def _sc_aggregate(ei_flat, x, ones_h, z_acc, z_cnt):
  mesh = plsc.VectorSubcoreMesh(core_axis_name="c", subcore_axis_name="s")

  @functools.partial(
      pl.kernel,
      out_type=[
          jax.ShapeDtypeStruct((NC, N_PAD, D), jnp.float32),
          jax.ShapeDtypeStruct((NC, N_PAD, CW), jnp.float32),
      ],
      mesh=mesh,
      compiler_params=pltpu.CompilerParams(use_tc_tiling_on_sc=False),
      scratch_types=[
          pltpu.VMEM((K,), jnp.int32),
          pltpu.VMEM((K,), jnp.int32),
          pltpu.VMEM((K,), jnp.int32),
          pltpu.VMEM((K,), jnp.int32),
          pltpu.VMEM((KT,), jnp.int32),
          pltpu.VMEM((KT,), jnp.int32),
          pltpu.VMEM((K, D), jnp.float32),
          pltpu.VMEM((K, D), jnp.float32),
          pltpu.VMEM((K, CW), jnp.float32),
          pltpu.VMEM_SHARED((N_PAD, D), jnp.float32),
          pltpu.VMEM_SHARED((N_PAD, CW), jnp.float32),
          pltpu.SemaphoreType.DMA,
          pltpu.SemaphoreType.DMA,
          pltpu.SemaphoreType.DMA,
          pltpu.SemaphoreType.DMA,
          pltpu.SemaphoreType.DMA,
          pltpu.SemaphoreType.DMA,
          pltpu.SemaphoreType.DMA,
          pltpu.SemaphoreType.DMA,
          pltpu.SemaphoreType.DMA,
          pltpu.SemaphoreType.DMA,
      ],
  )
  def agg_kernel(ei_h, x_h, ones_hbm, zacc_h, zcnt_h,
                 part_o, cnt_o,
                 sv0, sv1, dv0, dv1, svt, dvt, r0, r1, ones_v, acc, cnt,
                 gs0, gs1, ss0, ss1, cs0, cs1, is0, is1, id0, id1):
    sv = [sv0, sv1]
    dv = [dv0, dv1]
    rows = [r0, r1]
    gs = [gs0, gs1]
    ss = [ss0, ss1]
    cs = [cs0, cs1]
    isem = [is0, is1]
    idsem = [id0, id1]

    c_ax = lax.axis_index("c")
    s_ax = lax.axis_index("s")
    wid = s_ax * NC + c_ax
    ebase = wid * EPW

    def src_slice(c):
      return ei_h.at[pl.ds(ebase + c * K, K)]

    def dst_slice(c):
      return ei_h.at[pl.ds(N_EDGES + ebase + c * K, K)]

    pltpu.sync_copy(src_slice(0), sv[0])
    pltpu.async_copy(x_h.at[sv[0]], rows[0], gs[0])
    pltpu.async_copy(dst_slice(0), dv[0], idsem[0])
    pltpu.async_copy(src_slice(1), sv[1], isem[1])
    pltpu.sync_copy(ones_hbm, ones_v)
    pltpu.sync_copy(zacc_h, acc.at[pl.ds(s_ax * RPT, RPT)])
    pltpu.sync_copy(zcnt_h, cnt.at[pl.ds(s_ax * RPT, RPT)])
    plsc.subcore_barrier()
    pltpu.make_async_copy(x_h.at[sv[0]], rows[0], gs[0]).wait()
    pltpu.make_async_copy(dst_slice(0), dv[0], idsem[0]).wait()
    pltpu.async_copy(rows[0], acc.at[dv[0]], ss[0], add=True)
    pltpu.async_copy(ones_v, cnt.at[dv[0]], cs[0], add=True)
    pltpu.async_copy(dst_slice(1), dv[1], idsem[1])
    pltpu.async_copy(src_slice(2), sv[0], isem[0])
    pltpu.make_async_copy(src_slice(1), sv[1], isem[1]).wait()
    pltpu.async_copy(x_h.at[sv[1]], rows[1], gs[1])

    def chunk_step(c, b, bp):
      pltpu.make_async_copy(x_h.at[sv[b]], rows[b], gs[b]).wait()

      def _prefetch_src():
        pltpu.async_copy(src_slice(c + 2), sv[b], isem[b])

      if isinstance(c, int):
        if c + 2 < NCHUNK:
          _prefetch_src()
      else:
        pl.when(c + 2 < NCHUNK)(_prefetch_src)

      pltpu.make_async_copy(dst_slice(c), dv[b], idsem[b]).wait()
      pltpu.async_copy(rows[b], acc.at[dv[b]], ss[b], add=True)
      pltpu.async_copy(ones_v, cnt.at[dv[b]], cs[b], add=True)
      pltpu.make_async_copy(rows[bp], acc.at[dv[bp]], ss[bp]).wait()
      pltpu.make_async_copy(ones_v, cnt.at[dv[bp]], cs[bp]).wait()

      def _next_gather():
        pltpu.async_copy(dst_slice(c + 1), dv[bp], idsem[bp])
        pltpu.make_async_copy(src_slice(c + 1), sv[bp], isem[bp]).wait()
        pltpu.async_copy(x_h.at[sv[bp]], rows[bp], gs[bp])

      if isinstance(c, int):
        if c + 1 < NCHUNK:
          _next_gather()
      else:
        pl.when(c + 1 < NCHUNK)(_next_gather)

    chunk_step(1, 1, 0)

    @pl.loop(2, NCHUNK, step=2)
    def _body(g):
      for db in (0, 1):
        chunk_step(g + db, db, 1 - db)

    tbase = ebase + NCHUNK * K
    pltpu.sync_copy(ei_h.at[pl.ds(tbase, KT)], svt)
    pltpu.sync_copy(ei_h.at[pl.ds(N_EDGES + tbase, KT)], dvt)
    rows_t = rows[0].at[pl.ds(0, KT)]
    ones_t = ones_v.at[pl.ds(0, KT)]
    pltpu.async_copy(x_h.at[svt], rows_t, gs[0]).wait()
    pltpu.async_copy(rows_t, acc.at[dvt], ss[0], add=True)
    pltpu.async_copy(ones_t, cnt.at[dvt], cs[0], add=True)
    pltpu.make_async_copy(rows_t, acc.at[dvt], ss[0]).wait()
    pltpu.make_async_copy(ones_t, cnt.at[dvt], cs[0]).wait()
    pltpu.make_async_copy(rows[1], acc.at[dv[1]], ss[1]).wait()
    pltpu.make_async_copy(ones_v, cnt.at[dv[1]], cs[1]).wait()

    plsc.subcore_barrier()
    pltpu.sync_copy(acc.at[pl.ds(s_ax * RPT, RPT)],
                    part_o.at[c_ax, pl.ds(s_ax * RPT, RPT)])
    pltpu.sync_copy(cnt.at[pl.ds(s_ax * RPT, RPT)],
                    cnt_o.at[c_ax, pl.ds(s_ax * RPT, RPT)])

  return agg_kernel(ei_flat, x, ones_h, z_acc, z_cnt)


def _tc_root(x, W_r):
  R = 1000

  def body(x_ref, wr_ref, o_ref):
    dn = (((1,), (1,)), ((), ()))
    o_ref[...] = lax.dot_general(x_ref[...], wr_ref[...], dn,
                                 preferred_element_type=jnp.float32)

  return pl.pallas_call(
      body,
      grid=(N_NODES // R,),
      in_specs=[
          pl.BlockSpec((R, D), lambda i: (i, 0)),
          pl.BlockSpec((D, D), lambda i: (0, 0)),
      ],
      out_specs=pl.BlockSpec((R, D), lambda i: (i, 0)),
      out_shape=jax.ShapeDtypeStruct((N_NODES, D), jnp.float32),
  )(x, W_r)


def _tc_combine(part, cnt, x, root, W_l, b_l):
  R = 1000

  def body(part_ref, cnt_ref, x_ref, root_ref, wl_ref, bl_ref, o_ref):
    p = part_ref[0] + part_ref[1]
    deg = cnt_ref[0, :, 0:1] + cnt_ref[1, :, 0:1]
    agg = p / jnp.maximum(deg, 1.0)
    xb = x_ref[...]
    dn = (((1,), (1,)), ((), ()))
    h = (lax.dot_general(agg, wl_ref[...], dn,
                         preferred_element_type=jnp.float32)
         + bl_ref[...]
         + root_ref[...])
    o_ref[...] = xb + jnp.maximum(h, 0.0)

  return pl.pallas_call(
      body,
      grid=(N_NODES // R,),
      in_specs=[
          pl.BlockSpec((NC, R, D), lambda i: (0, i, 0)),
          pl.BlockSpec((NC, R, CW), lambda i: (0, i, 0)),
          pl.BlockSpec((R, D), lambda i: (i, 0)),
          pl.BlockSpec((R, D), lambda i: (i, 0)),
          pl.BlockSpec((D, D), lambda i: (0, 0)),
          pl.BlockSpec((1, D), lambda i: (0, 0)),
      ],
      out_specs=pl.BlockSpec((R, D), lambda i: (i, 0)),
      out_shape=jax.ShapeDtypeStruct((N_NODES, D), jnp.float32),
  )(part, cnt, x, root, W_l, b_l)


@jax.jit
def kernel(x, edge_index, W_l, b_l, W_r):
  ei_flat = edge_index.astype(jnp.int32).ravel()
  ones_h = jnp.ones((K, CW), jnp.float32)
  z_acc = jnp.zeros((RPT, D), jnp.float32)
  z_cnt = jnp.zeros((RPT, CW), jnp.float32)
  root = _tc_root(x, W_r)
  part, cnt = _sc_aggregate(ei_flat, x, ones_h, z_acc, z_cnt)
  return _tc_combine(part, cnt, x, root, W_l, b_l.reshape(1, D))
